# baseline (speedup 1.0000x reference)
import functools

import jax
import jax.numpy as jnp
from jax import lax
from jax.experimental import pallas as pl
from jax.experimental.pallas import tpu as pltpu
from jax.experimental.pallas import tpu_sc as plsc

_LANES = 16


def _sc_bias_rows_body(rows_per_w, n_sel, nc, sel_hbm, bias_hbm, out_hbm,
                       idx_v, bias_v, chunk_v):
    wid = lax.axis_index("s") * nc + lax.axis_index("c")
    pltpu.sync_copy(sel_hbm, idx_v)
    pltpu.sync_copy(bias_hbm, bias_v)
    bias_vec = bias_v[...]
    zeros = jnp.zeros((_LANES,), jnp.float32)

    def zero_body(i, carry):
        chunk_v[pl.ds(i * _LANES, _LANES)] = zeros
        return carry

    lax.fori_loop(0, rows_per_w // _LANES, zero_body, 0)

    base = wid * rows_per_w

    def scat_body(i, carry):
        idx = idx_v[pl.ds(i * _LANES, _LANES)]
        in_slab = (idx >= base) & (idx < base + rows_per_w)
        loc = jnp.where(in_slab, idx - base, 0)
        plsc.store_scatter(chunk_v, [loc], bias_vec, mask=in_slab)
        return carry

    lax.fori_loop(0, n_sel // _LANES, scat_body, 0)
    pltpu.sync_copy(chunk_v, out_hbm.at[pl.ds(base, rows_per_w)])


def _tc_add_body(data_ref, bias_ref, out_ref):
    b = bias_ref[...]
    out_ref[...] = data_ref[...] + lax.broadcast_in_dim(
        b, data_ref.shape, (0,))


@jax.jit
def kernel(data, selection, bias):
    n, d = data.shape
    n_sel = selection.shape[0]
    info = plsc.get_sparse_core_info()
    nw = info.num_cores * info.num_subcores
    rows_per_w = n // nw

    bias16 = jnp.full((_LANES,), bias, dtype=jnp.float32)
    sc_call = pl.kernel(
        functools.partial(_sc_bias_rows_body, rows_per_w, n_sel,
                          info.num_cores),
        out_type=jax.ShapeDtypeStruct((n,), jnp.float32),
        mesh=plsc.VectorSubcoreMesh(core_axis_name="c", subcore_axis_name="s"),
        scratch_types=[
            pltpu.VMEM((n_sel,), jnp.int32),
            pltpu.VMEM((_LANES,), jnp.float32),
            pltpu.VMEM((rows_per_w,), jnp.float32),
        ],
        compiler_params=pltpu.CompilerParams(needs_layout_passes=False),
    )
    bias_rows = sc_call(selection, bias16)

    br = 2048
    out = pl.pallas_call(
        _tc_add_body,
        out_shape=jax.ShapeDtypeStruct((n, d), jnp.float32),
        grid=(n // br,),
        in_specs=[
            pl.BlockSpec((br, d), lambda i: (i, 0)),
            pl.BlockSpec((br,), lambda i: (i,)),
        ],
        out_specs=pl.BlockSpec((br, d), lambda i: (i, 0)),
    )(data, bias_rows)
    return out


# speedup vs baseline: 1.0896x; 1.0896x over previous
import functools

import jax
import jax.numpy as jnp
from jax import lax
from jax.experimental import pallas as pl
from jax.experimental.pallas import tpu as pltpu
from jax.experimental.pallas import tpu_sc as plsc

_LANES = 16


def _sc_gather_add_body(spw, nc, data_hbm, sel_hbm, bias_hbm, fixed_hbm,
                        idx_v, bias_v, rows_v, sem):
    wid = lax.axis_index("s") * nc + lax.axis_index("c")
    base = wid * spw
    pltpu.sync_copy(sel_hbm.at[pl.ds(base, spw)], idx_v)
    pltpu.sync_copy(bias_hbm, bias_v)
    pltpu.async_copy(data_hbm.at[idx_v], rows_v, sem).wait()
    bias_vec = bias_v[...]
    nslice = rows_v.shape[1] // _LANES

    @pl.loop(0, spw)
    def _add(r):
        for c in range(nslice):
            sl = pl.ds(c * _LANES, _LANES)
            rows_v[r, sl] = rows_v[r, sl] + bias_vec

    pltpu.sync_copy(rows_v, fixed_hbm.at[pl.ds(base, spw)])


def _sc_scatter_body(spw, nc, sel_hbm, fixed_hbm, out_ref,
                     idx_v, rows_v, sem):
    wid = lax.axis_index("s") * nc + lax.axis_index("c")
    base = wid * spw
    pltpu.sync_copy(sel_hbm.at[pl.ds(base, spw)], idx_v)
    pltpu.sync_copy(fixed_hbm.at[pl.ds(base, spw)], rows_v)
    pltpu.async_copy(rows_v, out_ref.at[idx_v], sem).wait()


def _tc_copy_body(data_ref, out_ref):
    out_ref[...] = data_ref[...]


@jax.jit
def kernel(data, selection, bias):
    n, d = data.shape
    n_sel = selection.shape[0]
    info = plsc.get_sparse_core_info()
    nc = info.num_cores
    nw = nc * info.num_subcores
    spw = n_sel // nw

    mesh = plsc.VectorSubcoreMesh(core_axis_name="c", subcore_axis_name="s")
    sc_params = pltpu.CompilerParams(needs_layout_passes=False)

    bias16 = jnp.full((_LANES,), bias, dtype=jnp.float32)
    gather_add = pl.kernel(
        functools.partial(_sc_gather_add_body, spw, nc),
        out_type=jax.ShapeDtypeStruct((n_sel, d), jnp.float32),
        mesh=mesh,
        scratch_types=[
            pltpu.VMEM((spw,), jnp.int32),
            pltpu.VMEM((_LANES,), jnp.float32),
            pltpu.VMEM((spw, d), jnp.float32),
            pltpu.SemaphoreType.DMA,
        ],
        compiler_params=sc_params,
    )
    fixed_rows = gather_add(data, selection, bias16)

    br = 8192
    copied = pl.pallas_call(
        _tc_copy_body,
        out_shape=jax.ShapeDtypeStruct((n, d), jnp.float32),
        grid=(n // br,),
        in_specs=[pl.BlockSpec((br, d), lambda i: (i, 0))],
        out_specs=pl.BlockSpec((br, d), lambda i: (i, 0)),
    )(data)

    out_ref = jax.new_ref(copied)
    scatter = pl.kernel(
        functools.partial(_sc_scatter_body, spw, nc),
        out_type=(),
        mesh=mesh,
        scratch_types=[
            pltpu.VMEM((spw,), jnp.int32),
            pltpu.VMEM((spw, d), jnp.float32),
            pltpu.SemaphoreType.DMA,
        ],
        compiler_params=sc_params,
    )
    scatter(selection, fixed_rows, out_ref)
    return out_ref[...]


# speedup vs baseline: 1.0937x; 1.0037x over previous
import functools

import jax
import jax.numpy as jnp
from jax import lax
from jax.experimental import pallas as pl
from jax.experimental.pallas import tpu as pltpu
from jax.experimental.pallas import tpu_sc as plsc

_LANES = 16


def _sc_gather_add_body(spw, nc, data_hbm, sel_hbm, bias_hbm, fixed_hbm,
                        idx_v, bias_v, rows_v, sem):
    wid = lax.axis_index("s") * nc + lax.axis_index("c")
    base = wid * spw
    pltpu.sync_copy(sel_hbm.at[pl.ds(base, spw)], idx_v)
    pltpu.sync_copy(bias_hbm, bias_v)
    pltpu.async_copy(data_hbm.at[idx_v], rows_v, sem).wait()
    bias_vec = bias_v[...]
    nslice = rows_v.shape[1] // _LANES

    @pl.loop(0, spw, unroll=4)
    def _add(r):
        for c in range(nslice):
            sl = pl.ds(c * _LANES, _LANES)
            rows_v[r, sl] = rows_v[r, sl] + bias_vec

    pltpu.sync_copy(rows_v, fixed_hbm.at[pl.ds(base, spw)])


def _sc_scatter_body(spw, nc, sel_hbm, fixed_hbm, out_ref,
                     idx_v, rows_v, sem):
    wid = lax.axis_index("s") * nc + lax.axis_index("c")
    base = wid * spw
    pltpu.sync_copy(sel_hbm.at[pl.ds(base, spw)], idx_v)
    pltpu.sync_copy(fixed_hbm.at[pl.ds(base, spw)], rows_v)
    pltpu.async_copy(rows_v, out_ref.at[idx_v], sem).wait()


def _tc_copy_body(data_ref, out_ref):
    out_ref[...] = data_ref[...]


@jax.jit
def kernel(data, selection, bias):
    n, d = data.shape
    n_sel = selection.shape[0]
    info = plsc.get_sparse_core_info()
    nc = info.num_cores
    nw = nc * info.num_subcores
    spw = n_sel // nw

    mesh = plsc.VectorSubcoreMesh(core_axis_name="c", subcore_axis_name="s")
    sc_params = pltpu.CompilerParams(needs_layout_passes=False)

    bias16 = jnp.full((_LANES,), bias, dtype=jnp.float32)
    gather_add = pl.kernel(
        functools.partial(_sc_gather_add_body, spw, nc),
        out_type=jax.ShapeDtypeStruct((n_sel, d), jnp.float32),
        mesh=mesh,
        scratch_types=[
            pltpu.VMEM((spw,), jnp.int32),
            pltpu.VMEM((_LANES,), jnp.float32),
            pltpu.VMEM((spw, d), jnp.float32),
            pltpu.SemaphoreType.DMA,
        ],
        compiler_params=sc_params,
    )
    fixed_rows = gather_add(data, selection, bias16)

    br = 8192
    copied = pl.pallas_call(
        _tc_copy_body,
        out_shape=jax.ShapeDtypeStruct((n, d), jnp.float32),
        grid=(n // br,),
        in_specs=[pl.BlockSpec((br, d), lambda i: (i, 0))],
        out_specs=pl.BlockSpec((br, d), lambda i: (i, 0)),
    )(data)

    out_ref = jax.new_ref(copied)
    scatter = pl.kernel(
        functools.partial(_sc_scatter_body, spw, nc),
        out_type=(),
        mesh=mesh,
        scratch_types=[
            pltpu.VMEM((spw,), jnp.int32),
            pltpu.VMEM((spw, d), jnp.float32),
            pltpu.SemaphoreType.DMA,
        ],
        compiler_params=sc_params,
    )
    scatter(selection, fixed_rows, out_ref)
    return out_ref[...]
